# baseline (speedup 1.0000x reference)
import jax
import jax.numpy as jnp
from jax.experimental import pallas as pl
from jax.experimental.pallas import tpu as pltpu

INPUT_SIZE = 16384
NUM_COLUMNS = 2048
CELLS_PER_COLUMN = 32
SYN_PERM_CONNECTED = 0.14
NUM_ACTIVE = 90

CHUNK_C = 32
NCHUNK = NUM_COLUMNS // CHUNK_C
NBUF = 8
KEY_HI = 1 << 26


def _htm_kernel(x_ref, boost_ref, perm_hbm, out_ref, buf, keys_ref, sem):
    maskf = (x_ref[...] != 0).astype(jnp.float32)

    def copy(idx, b):
        return pltpu.make_async_copy(
            perm_hbm.at[pl.ds(idx * CHUNK_C, CHUNK_C), :], buf.at[b], sem.at[b]
        )

    for b in range(NBUF):
        copy(b, b).start()

    def process(idx, b):
        copy(idx, b).wait()
        p = buf[b]
        conn = (p >= SYN_PERM_CONNECTED).astype(jnp.float32)
        overlaps = jax.lax.dot_general(
            conn, maskf,
            dimension_numbers=(((1,), (1,)), ((), ())),
            preferred_element_type=jnp.float32,
        )
        boosted = overlaps * boost_ref[pl.ds(idx * CHUNK_C, CHUNK_C), :]
        col_ids = idx * CHUNK_C + jax.lax.broadcasted_iota(
            jnp.int32, (CHUNK_C, 1), 0
        )
        keys = boosted.astype(jnp.int32) * NUM_COLUMNS + (NUM_COLUMNS - 1 - col_ids)
        keys_ref[pl.ds(idx * CHUNK_C, CHUNK_C), :] = keys

        @pl.when(idx + NBUF < NCHUNK)
        def _():
            copy(idx + NBUF, b).start()

    def group(g, _):
        for b in range(NBUF):
            process(g * NBUF + b, b)
        return 0

    jax.lax.fori_loop(0, NCHUNK // NBUF, group, 0)

    all_keys = keys_ref[...]

    def body(_, carry):
        lo, hi = carry
        mid = (lo + hi) // 2
        cnt = jnp.sum((all_keys >= mid).astype(jnp.int32))
        big = cnt >= NUM_ACTIVE
        return jnp.where(big, mid, lo), jnp.where(big, hi, mid)

    lo, _ = jax.lax.fori_loop(0, 26, body, (jnp.int32(0), jnp.int32(KEY_HI)))
    active = (all_keys >= lo).astype(jnp.float32)
    out_ref[...] = jnp.broadcast_to(active, (NUM_COLUMNS, CELLS_PER_COLUMN))


@jax.jit
def _htm(x, permanences, boost):
    cells = pl.pallas_call(
        _htm_kernel,
        in_specs=[
            pl.BlockSpec((1, INPUT_SIZE), lambda: (0, 0)),
            pl.BlockSpec((NUM_COLUMNS, 1), lambda: (0, 0)),
            pl.BlockSpec(memory_space=pl.ANY),
        ],
        out_specs=pl.BlockSpec((NUM_COLUMNS, CELLS_PER_COLUMN), lambda: (0, 0)),
        out_shape=jax.ShapeDtypeStruct((NUM_COLUMNS, CELLS_PER_COLUMN), jnp.float32),
        scratch_shapes=[
            pltpu.VMEM((NBUF, CHUNK_C, INPUT_SIZE), jnp.float32),
            pltpu.VMEM((NUM_COLUMNS, 1), jnp.int32),
            pltpu.SemaphoreType.DMA((NBUF,)),
        ],
    )(x.reshape(1, INPUT_SIZE), boost.reshape(NUM_COLUMNS, 1), permanences)
    return cells.reshape(NUM_COLUMNS * CELLS_PER_COLUMN)


def kernel(x, permanences, boost):
    active_cells = _htm(x, permanences, boost)
    return active_cells, jnp.float32(1.0)


# speedup vs baseline: 1.0079x; 1.0079x over previous
import jax
import jax.numpy as jnp
from jax.experimental import pallas as pl
from jax.experimental.pallas import tpu as pltpu

INPUT_SIZE = 16384
NUM_COLUMNS = 2048
CELLS_PER_COLUMN = 32
SYN_PERM_CONNECTED = 0.14
NUM_ACTIVE = 90

CHUNK_C = 64
NCHUNK = NUM_COLUMNS // CHUNK_C
NBUF = 4
KEY_HI = 1 << 26


def _htm_kernel(x_ref, boost_ref, perm_hbm, out_ref, buf, keys_ref, sem):
    maskf = (x_ref[...] != 0).astype(jnp.float32)

    def copy(idx, b):
        return pltpu.make_async_copy(
            perm_hbm.at[pl.ds(idx * CHUNK_C, CHUNK_C), :], buf.at[b], sem.at[b]
        )

    for b in range(NBUF):
        copy(b, b).start()

    def process(idx, b):
        copy(idx, b).wait()
        p = buf[b]
        conn = (p >= SYN_PERM_CONNECTED).astype(jnp.float32)
        overlaps = jax.lax.dot_general(
            conn, maskf,
            dimension_numbers=(((1,), (1,)), ((), ())),
            preferred_element_type=jnp.float32,
        )
        boosted = overlaps * boost_ref[pl.ds(idx * CHUNK_C, CHUNK_C), :]
        col_ids = idx * CHUNK_C + jax.lax.broadcasted_iota(
            jnp.int32, (CHUNK_C, 1), 0
        )
        keys = boosted.astype(jnp.int32) * NUM_COLUMNS + (NUM_COLUMNS - 1 - col_ids)
        keys_ref[pl.ds(idx * CHUNK_C, CHUNK_C), :] = keys

        @pl.when(idx + NBUF < NCHUNK)
        def _():
            copy(idx + NBUF, b).start()

    def group(g, _):
        for b in range(NBUF):
            process(g * NBUF + b, b)
        return 0

    jax.lax.fori_loop(0, NCHUNK // NBUF, group, 0)

    all_keys = keys_ref[...]

    def body(_, carry):
        lo, hi = carry
        mid = (lo + hi) // 2
        cnt = jnp.sum((all_keys >= mid).astype(jnp.int32))
        big = cnt >= NUM_ACTIVE
        return jnp.where(big, mid, lo), jnp.where(big, hi, mid)

    lo, _ = jax.lax.fori_loop(0, 26, body, (jnp.int32(0), jnp.int32(KEY_HI)))
    active = (all_keys >= lo).astype(jnp.float32)
    out_ref[...] = jnp.broadcast_to(active, (NUM_COLUMNS, CELLS_PER_COLUMN))


@jax.jit
def _htm(x, permanences, boost):
    cells = pl.pallas_call(
        _htm_kernel,
        in_specs=[
            pl.BlockSpec((1, INPUT_SIZE), lambda: (0, 0)),
            pl.BlockSpec((NUM_COLUMNS, 1), lambda: (0, 0)),
            pl.BlockSpec(memory_space=pl.ANY),
        ],
        out_specs=pl.BlockSpec((NUM_COLUMNS, CELLS_PER_COLUMN), lambda: (0, 0)),
        out_shape=jax.ShapeDtypeStruct((NUM_COLUMNS, CELLS_PER_COLUMN), jnp.float32),
        scratch_shapes=[
            pltpu.VMEM((NBUF, CHUNK_C, INPUT_SIZE), jnp.float32),
            pltpu.VMEM((NUM_COLUMNS, 1), jnp.int32),
            pltpu.SemaphoreType.DMA((NBUF,)),
        ],
    )(x.reshape(1, INPUT_SIZE), boost.reshape(NUM_COLUMNS, 1), permanences)
    return cells.reshape(NUM_COLUMNS * CELLS_PER_COLUMN)


def kernel(x, permanences, boost):
    active_cells = _htm(x, permanences, boost)
    return active_cells, jnp.float32(1.0)
